# baseline (speedup 1.0000x reference)
import numpy as np
import jax
import jax.numpy as jnp
from jax.experimental import pallas as pl


def _i32(v):
    v &= 0xFFFFFFFF
    return v - (1 << 32) if v >= (1 << 31) else v


_ROTS = ((13, 15, 26, 6), (17, 29, 16, 24))


def _key_consts(k0, k1):
    ks = (k0, k1, k0 ^ k1 ^ 0x1BD11BDA)
    inj0 = [_i32(ks[(d + 1) % 3]) for d in range(5)]
    inj1 = [_i32(ks[(d + 2) % 3] + d + 1) for d in range(5)]
    return _i32(ks[0]), _i32(ks[1]), inj0, inj1


_MASK_KEY = _key_consts(64467757, 2916123636)
_PERT_KEY = _key_consts(2853785955, 313133857)

_CUTS = np.array(
    [4487996, 4521007, 4563722, 4621437, 4704320, 4834848, 5075329, 5692526],
    dtype=np.int32,
)
_CUTS_ROW = np.tile(_CUTS, 128).reshape(1, 1024)

_DRAW = np.float32(0.5302608013153076)

_ROWS = 2048
_COLS = 1024
_BLK = 256


def _rotl(x, r):
    return jax.lax.bitwise_or(
        jax.lax.shift_left(x, np.int32(r)),
        jax.lax.shift_right_logical(x, np.int32(32 - r)),
    )


def _threefry_bits(x1c, key_consts):
    ks0, ks1, inj0, inj1 = key_consts
    x1 = x1c + np.int32(ks1)
    x0 = x1 + np.int32(ks0)
    first = True
    for d in range(5):
        for r in _ROTS[d % 2]:
            if first:
                first = False
            else:
                x0 = x0 + x1
            x1 = jax.lax.bitwise_xor(_rotl(x1, r), x0)
        x0 = x0 + np.int32(inj0[d])
        x1 = x1 + np.int32(inj1[d])
    return jax.lax.bitwise_xor(x0, x1)


def _perturb_body(codes_ref, cuts_ref, out_ref):
    g = pl.program_id(0)
    base = g * np.int32(_BLK * _COLS)
    rows = jax.lax.broadcasted_iota(jnp.int32, (_BLK, _COLS), 0)
    cols = jax.lax.broadcasted_iota(jnp.int32, (_BLK, _COLS), 1)
    f = base + rows * np.int32(_COLS) + cols
    i = jax.lax.shift_left(jax.lax.bitwise_and(f, np.int32(7)), np.int32(18))
    i = i + jax.lax.shift_right_logical(f, np.int32(3))
    bm = _threefry_bits(i, _MASK_KEY)
    bp = _threefry_bits(i, _PERT_KEY)
    mantissa = jax.lax.shift_right_logical(bm, np.int32(9))
    mask = mantissa <= cuts_ref[...]
    pert = jax.lax.bitwise_and(bp, np.int32(2047))
    out_ref[...] = jnp.where(mask, pert, codes_ref[...])


def kernel(codes, perturb_prob=1):
    apply_gate = _DRAW <= jnp.float32(perturb_prob)
    cuts_eff = jnp.where(apply_gate, jnp.asarray(_CUTS_ROW), np.int32(-1))
    codes2d = codes.reshape(_ROWS, _COLS)
    out = pl.pallas_call(
        _perturb_body,
        grid=(_ROWS // _BLK,),
        in_specs=[
            pl.BlockSpec((_BLK, _COLS), lambda g: (g, 0)),
            pl.BlockSpec((1, _COLS), lambda g: (0, 0)),
        ],
        out_specs=pl.BlockSpec((_BLK, _COLS), lambda g: (g, 0)),
        out_shape=jax.ShapeDtypeStruct((_ROWS, _COLS), jnp.int32),
    )(codes2d, cuts_eff)
    return out.reshape(codes.shape)


# speedup vs baseline: 2.8855x; 2.8855x over previous
import numpy as np
import jax
import jax.numpy as jnp
from jax.experimental import pallas as pl
from jax.experimental.pallas import tpu as pltpu


def _i32(v):
    v &= 0xFFFFFFFF
    return v - (1 << 32) if v >= (1 << 31) else v


_ROTS = ((13, 15, 26, 6), (17, 29, 16, 24))


def _key_consts(k0, k1):
    ks = (k0, k1, k0 ^ k1 ^ 0x1BD11BDA)
    inj0 = [_i32(ks[(d + 1) % 3]) for d in range(5)]
    inj1 = [_i32(ks[(d + 2) % 3] + d + 1) for d in range(5)]
    return _i32(ks[0]), _i32(ks[1]), inj0, inj1


_MASK_KEY = _key_consts(64467757, 2916123636)
_PERT_KEY = _key_consts(2853785955, 313133857)

_CUTS = np.array(
    [4487996, 4521007, 4563722, 4621437, 4704320, 4834848, 5075329, 5692526],
    dtype=np.int32,
)

_CUTS_COL = np.tile(_CUTS, 32).reshape(256, 1)

_ROW_BASE = (
    (np.arange(256, dtype=np.int64) % 8) * 262144
    + (np.arange(256, dtype=np.int64) // 8) * 8192
).astype(np.int32).reshape(256, 1)

_DRAW = np.float32(0.5302608013153076)

_NROWS = 256
_NCOLS = 8192
_BLKC = 2048
_CHUNK = 256


def _rotl(x, r):
    return jax.lax.bitwise_or(
        jax.lax.shift_left(x, np.int32(r)),
        jax.lax.shift_right_logical(x, np.int32(32 - r)),
    )


def _threefry_bits(x1c, key_consts):
    ks0, ks1, inj0, inj1 = key_consts
    x1 = x1c + np.int32(ks1)
    x0 = x1 + np.int32(ks0)
    first = True
    for d in range(5):
        for r in _ROTS[d % 2]:
            if first:
                first = False
            else:
                x0 = x0 + x1
            x1 = jax.lax.bitwise_xor(_rotl(x1, r), x0)
        x0 = x0 + np.int32(inj0[d])
        x1 = x1 + np.int32(inj1[d])
    return jax.lax.bitwise_xor(x0, x1)


def _perturb_body(codes_ref, cuts_ref, rowb_ref, out_ref):
    g = pl.program_id(0)
    cuts = cuts_ref[...]
    row_base = rowb_ref[...]
    cols = jax.lax.broadcasted_iota(jnp.int32, (_NROWS, _CHUNK), 1)
    i0 = row_base + cols

    def step(c, _):
        i = i0 + (g * np.int32(_BLKC) + c * np.int32(_CHUNK))
        bm = _threefry_bits(i, _MASK_KEY)
        bp = _threefry_bits(i, _PERT_KEY)
        mantissa = jax.lax.shift_right_logical(bm, np.int32(9))
        mask = mantissa <= cuts
        pert = jax.lax.bitwise_and(bp, np.int32(2047))
        sl = pl.ds(c * _CHUNK, _CHUNK)
        out_ref[:, sl] = jnp.where(mask, pert, codes_ref[:, sl])
        return _

    jax.lax.fori_loop(0, _BLKC // _CHUNK, step, 0)


def kernel(codes, perturb_prob=1):
    apply_gate = _DRAW <= jnp.float32(perturb_prob)
    cuts_eff = jnp.where(apply_gate, jnp.asarray(_CUTS_COL), np.int32(-1))
    codes2d = jnp.transpose(codes, (0, 2, 1)).reshape(_NROWS, _NCOLS)
    out = pl.pallas_call(
        _perturb_body,
        grid=(_NCOLS // _BLKC,),
        in_specs=[
            pl.BlockSpec((_NROWS, _BLKC), lambda g: (0, g)),
            pl.BlockSpec((_NROWS, 1), lambda g: (0, 0)),
            pl.BlockSpec((_NROWS, 1), lambda g: (0, 0)),
        ],
        out_specs=pl.BlockSpec((_NROWS, _BLKC), lambda g: (0, g)),
        out_shape=jax.ShapeDtypeStruct((_NROWS, _NCOLS), jnp.int32),
        compiler_params=pltpu.CompilerParams(
            dimension_semantics=("parallel",),
        ),
    )(codes2d, cuts_eff, jnp.asarray(_ROW_BASE))
    return jnp.transpose(out.reshape(32, 8, 8192), (0, 2, 1))
